# Initial kernel scaffold; baseline (speedup 1.0000x reference)
import jax
import jax.numpy as jnp
from jax.experimental import pallas as pl


def kernel(user, item, user_edge_index, item_edge_index, item_users, ii_sim_users, ii_sim_lens, user_items, uu_sim_items, uu_sim_lens, params):
    raise NotImplementedError("write your pallas kernel here")


# speedup vs baseline: 1.0001x; 1.0001x over previous
import functools

import jax
import jax.numpy as jnp
import numpy as np
from jax.experimental import pallas as pl
from jax.experimental.pallas import tpu as pltpu

NUM_USER = 50000
NUM_ITEM = 50000
D = 64
NUM_LAYER = 2
B = 1024
E = 800000
L_HIST = 20
S_SIM = 10
L_SIM = 20
S_DIM = 48


def _aggregate(edge_index, x, n):
    src = edge_index[0]
    dst = edge_index[1]
    msg = jnp.take(x, src, axis=0)
    agg = jax.ops.segment_sum(msg, dst, num_segments=n)
    deg = jax.ops.segment_sum(jnp.ones((dst.shape[0],), x.dtype), dst, num_segments=n)
    return agg / jnp.maximum(deg, 1.0)[:, None]


def _graph_aggregate(edge_index, x, n):
    init = x
    all_e = [init]
    for _ in range(NUM_LAYER):
        init = _aggregate(edge_index, init, n)
        nrm = jnp.maximum(jnp.linalg.norm(init, axis=1, keepdims=True), 1e-12)
        all_e.append(init / nrm)
    return jnp.mean(jnp.stack(all_e, 0), axis=0)


def _mha(q_in, k_in, v_in, Wq, Wk, Wv, Wo, h):
    L, Bq, d = q_in.shape
    Lk = k_in.shape[0]
    dh = d // h
    q = (q_in @ Wq).reshape(L, Bq, h, dh)
    k = (k_in @ Wk).reshape(Lk, Bq, h, dh)
    v = (v_in @ Wv).reshape(Lk, Bq, h, dh)
    s = jnp.einsum('lbhd,mbhd->bhlm', q, k) / np.sqrt(dh).astype(np.float32)
    a = jax.nn.softmax(s, axis=-1)
    o = jnp.einsum('bhlm,mbhd->lbhd', a, v).reshape(L, Bq, d)
    return o @ Wo


def _sim_attention(active, f_embed, lens, W):
    s = jnp.einsum('bd,bsd->bs', active @ W, f_embed) / np.sqrt(D).astype(np.float32)
    mask = jnp.arange(f_embed.shape[1])[None, :] < lens[:, None]
    s = jnp.where(mask, s, -1e9)
    return jax.nn.softmax(s, axis=-1)


def _ffn(x, x_, Wf1, bf1, Wf2, bf2):
    h = x + x_
    return h + jax.nn.relu(h @ Wf1 + bf1) @ Wf2 + bf2


def _pred_body(ue_ref, ie_ref, ui_ref, ia_ref, w_ref, b1_ref, w2_ref, out_ref):
    ue = ue_ref[...]
    ie = ie_ref[...]
    ui = ui_ref[...]
    ia = ia_ref[...]
    lefts = (ue, ui, ue, ui)
    rights = (ie, ie, ia, ia)
    cols = []
    for k in range(4):
        h = (lefts[k] @ w_ref[k, 0] + rights[k] @ w_ref[k, 1]
             + b1_ref[k][None, :])
        h = jnp.maximum(h, 0.0)
        s = jnp.sum(h * w2_ref[k, :48][None, :], axis=1, keepdims=True)
        cols.append(s)
    out_ref[...] = jnp.concatenate(cols, axis=1)


@jax.jit
def _predictors(ue, ie, ui, ia, w1s, b1s, w2s):
    return pl.pallas_call(
        _pred_body,
        out_shape=jax.ShapeDtypeStruct((B, 4), jnp.float32),
    )(ue, ie, ui, ia, w1s, b1s, w2s)


def kernel(user, item, user_edge_index, item_edge_index, item_users,
           ii_sim_users, ii_sim_lens, user_items, uu_sim_items, uu_sim_lens,
           params):
    p = params
    nU = NUM_USER + 1
    nI = NUM_ITEM + 1
    user_pref = _graph_aggregate(user_edge_index, p['user_table'], nU)
    user_pref = user_pref.at[0].set(0.0)
    item_attr = _graph_aggregate(item_edge_index, p['item_table'], nI)
    item_attr = item_attr.at[0].set(0.0)
    u_e = jnp.take(user_pref, user, axis=0)
    i_e = jnp.take(item_attr, item, axis=0)
    iu = jnp.take(user_pref, item_users, axis=0)
    item_attraction = jnp.max(iu * u_e[:, None, :], axis=1)
    f = jnp.take(user_pref, ii_sim_users, axis=0)
    f = jnp.max(f * u_e[:, None, None, :], axis=2)
    a = _sim_attention(item_attraction, f, ii_sim_lens, p['W_ii'])
    item_neigh = jnp.einsum('bs,bsd->bd', a, f)
    item_attraction = 0.5 * (item_attraction + item_neigh)
    x = jnp.take(item_attr, user_items, axis=0)
    x = jnp.transpose(x, (1, 0, 2))
    x_ = _mha(x, x, x, p['Wq1'], p['Wk1'], p['Wv1'], p['Wo1'], 2)
    x = _ffn(x, x_, p['Wf1'], p['bf1'], p['Wf2'], p['bf2'])
    t = i_e[None, :, :]
    user_interest = _mha(t, x, x, p['Wq2'], p['Wk2'], p['Wv2'], p['Wo2'], 1)[0]
    nf = jnp.take(item_attr, uu_sim_items, axis=0)
    nf = jnp.max(nf * i_e[:, None, None, :], axis=2)
    a2 = _sim_attention(user_interest, nf, uu_sim_lens, p['W_uu'])
    user_neigh = jnp.einsum('bs,bsd->bd', a2, nf)
    user_interest = 0.5 * (user_interest + user_neigh)

    w1s = jnp.stack([p['P%d_W1' % k].reshape(2, D, S_DIM) for k in (1, 2, 3, 4)])
    b1s = jnp.stack([p['P%d_b1' % k] for k in (1, 2, 3, 4)])
    b2s = jnp.stack([p['P%d_b2' % k][0] for k in (1, 2, 3, 4)])
    w2s = jnp.stack([jnp.pad(p['P%d_W2' % k][:, 0], (0, D - S_DIM))
                     for k in (1, 2, 3, 4)])
    scores = _predictors(u_e, i_e, user_interest, item_attraction,
                         w1s, b1s, w2s)
    return scores + b2s[None, :]


# speedup vs baseline: 6.5560x; 6.5556x over previous
import functools

import jax
import jax.numpy as jnp
import numpy as np
from jax import lax
from jax.experimental import pallas as pl
from jax.experimental.pallas import tpu as pltpu
from jax.experimental.pallas import tpu_sc as plsc

NUM_USER = 50000
NUM_ITEM = 50000
D = 64
B = 1024
E = 800000
L_HIST = 20
S_SIM = 10
L_SIM = 20
S_DIM = 48

N = NUM_USER + 1
NP = 50176
DUMP = N
RT = NP // 16

EB = 512
ECH = EB // 128
NBLK = 98
EPAD = 16 * NBLK * EB
EROWS = EPAD // 128

NEG = np.float32(-3.4e38)


def _mesh():
    return plsc.VectorSubcoreMesh(core_axis_name="c", subcore_axis_name="s")


_SC_PARAMS = pltpu.CompilerParams(use_tc_tiling_on_sc=False)


def _sc_degree(dsts2d):
    @functools.partial(
        pl.kernel, mesh=_mesh(),
        out_type=jax.ShapeDtypeStruct((2, NP, 16), jnp.float32),
        scratch_types=[
            pltpu.VMEM((ECH, 128), jnp.int32),
            pltpu.VMEM((EB, 16), jnp.float32),
            pltpu.VMEM((EB, 16), jnp.float32),
            pltpu.VMEM_SHARED((NP, 16), jnp.float32),
            pltpu.SemaphoreType.DMA,
        ],
        compiler_params=_SC_PARAMS,
    )
    def k(dst_hbm, out_hbm, dbuf, ones, zbuf, acc, ssem):
        cid = lax.axis_index("c")
        sid = lax.axis_index("s")

        def initrow(i, _):
            ones[i, :] = jnp.full((16,), 1.0, jnp.float32)
            zbuf[i, :] = jnp.full((16,), 0.0, jnp.float32)
            return 0

        lax.fori_loop(0, EB, initrow, 0)
        r0 = sid * RT
        for t in range(6):
            pltpu.sync_copy(zbuf, acc.at[pl.ds(r0 + t * EB, EB)])
        pltpu.sync_copy(zbuf.at[pl.ds(0, RT - 6 * EB)],
                        acc.at[pl.ds(r0 + 6 * EB, RT - 6 * EB)])
        plsc.subcore_barrier()

        def blk(i, _):
            base = (sid * NBLK + i) * ECH
            pltpu.sync_copy(dst_hbm.at[cid].at[pl.ds(base, ECH)], dbuf)
            sd = [pltpu.async_copy(ones.at[pl.ds(j * 128, 128)],
                                   acc.at[dbuf.at[j]], ssem, add=True)
                  for j in range(ECH)]
            for dsc in sd:
                dsc.wait()
            return 0

        lax.fori_loop(0, NBLK, blk, 0)
        plsc.subcore_barrier()
        pltpu.sync_copy(acc.at[pl.ds(r0, RT)],
                        out_hbm.at[cid].at[pl.ds(r0, RT)])

    return k(dsts2d)


def _sc_agg(x2, src2d, dst2d):
    @functools.partial(
        pl.kernel, mesh=_mesh(),
        out_type=jax.ShapeDtypeStruct((2, NP, 32), jnp.float32),
        scratch_types=[
            pltpu.VMEM((ECH, 128), jnp.int32),
            pltpu.VMEM((ECH, 128), jnp.int32),
            pltpu.VMEM((EB, 32), jnp.float32),
            pltpu.VMEM_SHARED((NP, 32), jnp.float32),
            pltpu.SemaphoreType.DMA,
            pltpu.SemaphoreType.DMA,
        ],
        compiler_params=_SC_PARAMS,
    )
    def k(x2_hbm, src_hbm, dst_hbm, out_hbm, sbuf, dbuf, rows, acc,
          gsem, ssem):
        cid = lax.axis_index("c")
        sid = lax.axis_index("s")

        def zrow(i, _):
            rows[i, 0:16] = jnp.full((16,), 0.0, jnp.float32)
            rows[i, 16:32] = jnp.full((16,), 0.0, jnp.float32)
            return 0

        lax.fori_loop(0, EB, zrow, 0)
        r0 = sid * RT
        for t in range(6):
            pltpu.sync_copy(rows, acc.at[pl.ds(r0 + t * EB, EB)])
        pltpu.sync_copy(rows.at[pl.ds(0, RT - 6 * EB)],
                        acc.at[pl.ds(r0 + 6 * EB, RT - 6 * EB)])
        plsc.subcore_barrier()

        def blk(i, _):
            base = (sid * NBLK + i) * ECH
            pltpu.sync_copy(src_hbm.at[pl.ds(base, ECH)], sbuf)
            pltpu.sync_copy(dst_hbm.at[pl.ds(base, ECH)], dbuf)
            gd = [pltpu.async_copy(x2_hbm.at[cid].at[sbuf.at[j]],
                                   rows.at[pl.ds(j * 128, 128)], gsem)
                  for j in range(ECH)]
            for dsc in gd:
                dsc.wait()
            sd = [pltpu.async_copy(rows.at[pl.ds(j * 128, 128)],
                                   acc.at[dbuf.at[j]], ssem, add=True)
                  for j in range(ECH)]
            for dsc in sd:
                dsc.wait()
            return 0

        lax.fori_loop(0, NBLK, blk, 0)
        plsc.subcore_barrier()
        pltpu.sync_copy(acc.at[pl.ds(r0, RT)],
                        out_hbm.at[cid].at[pl.ds(r0, RT)])

    return k(x2, src2d, dst2d)


def _sc_batch(up, it, user, item, iu2d, iiflat, ui2d, uuflat):
    bs = B // 32

    @functools.partial(
        pl.kernel, mesh=_mesh(),
        out_type=[
            jax.ShapeDtypeStruct((B, 64), jnp.float32),
            jax.ShapeDtypeStruct((B, 64), jnp.float32),
            jax.ShapeDtypeStruct((B, 64), jnp.float32),
            jax.ShapeDtypeStruct((B * S_SIM, 64), jnp.float32),
            jax.ShapeDtypeStruct((B * L_HIST, 64), jnp.float32),
            jax.ShapeDtypeStruct((B * S_SIM, 64), jnp.float32),
        ],
        scratch_types=[
            pltpu.VMEM((bs,), jnp.int32),
            pltpu.VMEM((bs,), jnp.int32),
            pltpu.VMEM((bs * L_HIST // 128, 128), jnp.int32),
            pltpu.VMEM((bs * L_SIM * S_SIM,), jnp.int32),
            pltpu.VMEM((bs, 64), jnp.float32),
            pltpu.VMEM((bs, 64), jnp.float32),
            pltpu.VMEM((bs * L_HIST, 64), jnp.float32),
            pltpu.VMEM((L_SIM * S_SIM, 64), jnp.float32),
            pltpu.VMEM((bs, 64), jnp.float32),
            pltpu.VMEM((bs * S_SIM, 64), jnp.float32),
            pltpu.VMEM((bs * S_SIM, 64), jnp.float32),
            pltpu.SemaphoreType.DMA,
        ],
        compiler_params=_SC_PARAMS,
    )
    def k(up_hbm, it_hbm, u_hbm, i_hbm, iu_hbm, ii_hbm, ui_hbm, uu_hbm,
          ue_out, ie_out, iap_out, f_out, xh_out, nf_out,
          ubuf, ibuf, hbuf, sbuf, uerows, ierows, hrows, srows,
          iap, fbuf, nfbuf, sem):
        cid = lax.axis_index("c")
        sid = lax.axis_index("s")
        wid = sid * 2 + cid
        gb0 = wid * bs
        hch = bs * L_HIST // 128

        pltpu.sync_copy(u_hbm.at[pl.ds(gb0, bs)], ubuf)
        pltpu.sync_copy(i_hbm.at[pl.ds(gb0, bs)], ibuf)
        pltpu.async_copy(up_hbm.at[ubuf], uerows, sem).wait()
        pltpu.async_copy(it_hbm.at[ibuf], ierows, sem).wait()

        pltpu.sync_copy(iu_hbm.at[pl.ds(wid * hch, hch)], hbuf)
        gd = [pltpu.async_copy(up_hbm.at[hbuf.at[j]],
                               hrows.at[pl.ds(j * 128, 128)], sem)
              for j in range(hch)]
        for dsc in gd:
            dsc.wait()

        def iab(b, _):
            for j in range(4):
                ue16 = uerows[b, pl.ds(j * 16, 16)]

                def lb(l, m):
                    r = hrows[b * L_HIST + l, pl.ds(j * 16, 16)]
                    return jnp.maximum(m, r * ue16)

                m = lax.fori_loop(0, L_HIST, lb,
                                  jnp.full((16,), NEG, jnp.float32))
                iap[b, pl.ds(j * 16, 16)] = m
            return 0

        lax.fori_loop(0, bs, iab, 0)

        pltpu.sync_copy(ui_hbm.at[pl.ds(wid * hch, hch)], hbuf)
        gd = [pltpu.async_copy(it_hbm.at[hbuf.at[j]],
                               hrows.at[pl.ds(j * 128, 128)], sem)
              for j in range(hch)]
        for dsc in gd:
            dsc.wait()
        pltpu.sync_copy(hrows, xh_out.at[pl.ds(gb0 * L_HIST, bs * L_HIST)])

        nsim = L_SIM * S_SIM
        pltpu.sync_copy(ii_hbm.at[pl.ds(gb0 * nsim, bs * nsim)], sbuf)

        def fb(b, _):
            pltpu.async_copy(
                up_hbm.at[sbuf.at[pl.ds(b * nsim, 128)]],
                srows.at[pl.ds(0, 128)], sem).wait()
            pltpu.async_copy(
                up_hbm.at[sbuf.at[pl.ds(b * nsim + 128, nsim - 128)]],
                srows.at[pl.ds(128, nsim - 128)], sem).wait()

            def sb(s, _2):
                for j in range(4):
                    ue16 = uerows[b, pl.ds(j * 16, 16)]

                    def lb(l, m):
                        r = srows[s * L_SIM + l, pl.ds(j * 16, 16)]
                        return jnp.maximum(m, r * ue16)

                    m = lax.fori_loop(0, L_SIM, lb,
                                      jnp.full((16,), NEG, jnp.float32))
                    fbuf[b * S_SIM + s, pl.ds(j * 16, 16)] = m
                return 0

            lax.fori_loop(0, S_SIM, sb, 0)
            return 0

        lax.fori_loop(0, bs, fb, 0)

        pltpu.sync_copy(uu_hbm.at[pl.ds(gb0 * nsim, bs * nsim)], sbuf)

        def nb(b, _):
            pltpu.async_copy(
                it_hbm.at[sbuf.at[pl.ds(b * nsim, 128)]],
                srows.at[pl.ds(0, 128)], sem).wait()
            pltpu.async_copy(
                it_hbm.at[sbuf.at[pl.ds(b * nsim + 128, nsim - 128)]],
                srows.at[pl.ds(128, nsim - 128)], sem).wait()

            def sb(s, _2):
                for j in range(4):
                    ie16 = ierows[b, pl.ds(j * 16, 16)]

                    def lb(l, m):
                        r = srows[s * L_SIM + l, pl.ds(j * 16, 16)]
                        return jnp.maximum(m, r * ie16)

                    m = lax.fori_loop(0, L_SIM, lb,
                                      jnp.full((16,), NEG, jnp.float32))
                    nfbuf[b * S_SIM + s, pl.ds(j * 16, 16)] = m
                return 0

            lax.fori_loop(0, S_SIM, sb, 0)
            return 0

        lax.fori_loop(0, bs, nb, 0)

        pltpu.sync_copy(uerows, ue_out.at[pl.ds(gb0, bs)])
        pltpu.sync_copy(ierows, ie_out.at[pl.ds(gb0, bs)])
        pltpu.sync_copy(iap, iap_out.at[pl.ds(gb0, bs)])
        pltpu.sync_copy(fbuf, f_out.at[pl.ds(gb0 * S_SIM, bs * S_SIM)])
        pltpu.sync_copy(nfbuf, nf_out.at[pl.ds(gb0 * S_SIM, bs * S_SIM)])

    return k(up, it, user, item, iu2d, iiflat, ui2d, uuflat)


def _tc_prep(tab_pad):
    def body(x_ref, o_ref):
        x = x_ref[...]
        o_ref[0] = x[:, :32]
        o_ref[1] = x[:, 32:]

    return pl.pallas_call(
        body,
        grid=(16,),
        in_specs=[pl.BlockSpec((RT, 64), lambda i: (i, 0))],
        out_specs=pl.BlockSpec((2, RT, 32), lambda i: (0, i, 0)),
        out_shape=jax.ShapeDtypeStruct((2, NP, 32), jnp.float32),
    )(tab_pad)


def _tc_norm(raw, deg16, prev, final):
    def body(raw_ref, deg_ref, prev_ref, *out_refs):
        i = pl.program_id(0)
        raw = raw_ref[...]
        d = jnp.maximum(deg_ref[:, 0:1], 1.0)
        a0 = raw[0] / d
        a1 = raw[1] / d
        nsq = (jnp.sum(a0 * a0, axis=1, keepdims=True)
               + jnp.sum(a1 * a1, axis=1, keepdims=True))
        inv = 1.0 / jnp.maximum(jnp.sqrt(nsq), 1e-12)
        n = jnp.concatenate([a0 * inv, a1 * inv], axis=1)
        if final:
            gi = i * RT + lax.broadcasted_iota(jnp.int32, (RT, 1), 0)
            out = (prev_ref[...] + n) * jnp.float32(1.0 / 3.0)
            out_refs[0][...] = jnp.where(gi == 0, 0.0, out)
        else:
            out_refs[0][0] = a0
            out_refs[0][1] = a1
            out_refs[1][...] = prev_ref[...] + n

    if final:
        out_shape = [jax.ShapeDtypeStruct((NP, 64), jnp.float32)]
        out_specs = [pl.BlockSpec((RT, 64), lambda i: (i, 0))]
    else:
        out_shape = [jax.ShapeDtypeStruct((2, NP, 32), jnp.float32),
                     jax.ShapeDtypeStruct((NP, 64), jnp.float32)]
        out_specs = [pl.BlockSpec((2, RT, 32), lambda i: (0, i, 0)),
                     pl.BlockSpec((RT, 64), lambda i: (i, 0))]
    res = pl.pallas_call(
        body,
        grid=(16,),
        in_specs=[pl.BlockSpec((2, RT, 32), lambda i: (0, i, 0)),
                  pl.BlockSpec((RT, 16), lambda i: (i, 0)),
                  pl.BlockSpec((RT, 64), lambda i: (i, 0))],
        out_specs=out_specs,
        out_shape=out_shape,
    )(raw, deg16, prev)
    return res[0] if final else res


def _tc_dense(ue, ie, iapre, f, xh, nf, lens_ii, lens_uu, w):
    BB = 128
    SQD = np.float32(1.0 / np.sqrt(D))
    SQH = np.float32(1.0 / np.sqrt(32))

    def body(ue_ref, ie_ref, iap_ref, f_ref, xh_ref, nf_ref, li_ref, lu_ref,
             wii_ref, wuu_ref, wq1_ref, wk1_ref, wv1_ref, wo1_ref,
             wq2_ref, wk2_ref, wv2_ref, wo2_ref, wf1_ref, bf1_ref,
             wf2_ref, bf2_ref, w1s_ref, b1s_ref, w2s_ref, b2s_ref, out_ref):
        uev = ue_ref[...]
        iev = ie_ref[...]
        iap = iap_ref[...]
        fv = f_ref[...]
        nfv = nf_ref[...]
        xhv = xh_ref[...]
        li = li_ref[...].reshape(BB)
        lu = lu_ref[...].reshape(BB)

        def sim_fuse(active, fe, lens, wmat):
            act = active @ wmat
            s = jnp.sum(act[:, None, :] * fe, axis=-1) * SQD
            mask = (lax.broadcasted_iota(jnp.int32, (BB, S_SIM), 1)
                    < lens[:, None])
            s = jnp.where(mask, s, -1e9)
            a = jax.nn.softmax(s, axis=-1)
            return jnp.sum(a[:, :, None] * fe, axis=1)

        item_neigh = sim_fuse(iap, fv, li, wii_ref[...])
        ia = 0.5 * (iap + item_neigh)

        xf = xhv.reshape(BB * L_HIST, D)
        q = (xf @ wq1_ref[...]).reshape(BB, L_HIST, D)
        kk = (xf @ wk1_ref[...]).reshape(BB, L_HIST, D)
        vv = (xf @ wv1_ref[...]).reshape(BB, L_HIST, D)
        outs = []
        for h in range(2):
            qh = q[:, :, h * 32:(h + 1) * 32]
            kh = kk[:, :, h * 32:(h + 1) * 32]
            vh = vv[:, :, h * 32:(h + 1) * 32]
            sh = lax.dot_general(qh, kh, (((2,), (2,)), ((0,), (0,))),
                                 preferred_element_type=jnp.float32) * SQH
            ah = jax.nn.softmax(sh, axis=-1)
            oh = lax.dot_general(ah, vh, (((2,), (1,)), ((0,), (0,))),
                                 preferred_element_type=jnp.float32)
            outs.append(oh)
        o = jnp.concatenate(outs, axis=-1).reshape(BB * L_HIST, D)
        x_ = o @ wo1_ref[...]
        h1 = xf + x_
        x = (h1 + jnp.maximum(h1 @ wf1_ref[...] + bf1_ref[...], 0.0)
             @ wf2_ref[...] + bf2_ref[...])
        x3 = x.reshape(BB, L_HIST, D)

        q2 = iev @ wq2_ref[...]
        k2 = (x @ wk2_ref[...]).reshape(BB, L_HIST, D)
        v2 = (x @ wv2_ref[...]).reshape(BB, L_HIST, D)
        s2 = jnp.sum(q2[:, None, :] * k2, axis=-1) * SQD
        a2 = jax.nn.softmax(s2, axis=-1)
        o2 = jnp.sum(a2[:, :, None] * v2, axis=1)
        uiv = o2 @ wo2_ref[...]

        user_neigh = sim_fuse(uiv, nfv, lu, wuu_ref[...])
        ui = 0.5 * (uiv + user_neigh)

        lefts = (uev, ui, uev, ui)
        rights = (iev, iev, ia, ia)
        cols = []
        for kq in range(4):
            hh = (lefts[kq] @ w1s_ref[kq, 0] + rights[kq] @ w1s_ref[kq, 1]
                  + b1s_ref[kq][None, :])
            hh = jnp.maximum(hh, 0.0)
            sc = jnp.sum(hh * w2s_ref[kq, :S_DIM][None, :], axis=1,
                         keepdims=True)
            cols.append(sc)
        out_ref[...] = jnp.concatenate(cols, axis=1) + b2s_ref[...]

    nb = B // BB
    full = lambda shape: pl.BlockSpec(shape, lambda i: tuple(0 for _ in shape))
    in_specs = [
        pl.BlockSpec((BB, 64), lambda i: (i, 0)),
        pl.BlockSpec((BB, 64), lambda i: (i, 0)),
        pl.BlockSpec((BB, 64), lambda i: (i, 0)),
        pl.BlockSpec((BB, S_SIM, 64), lambda i: (i, 0, 0)),
        pl.BlockSpec((BB, L_HIST, 64), lambda i: (i, 0, 0)),
        pl.BlockSpec((BB, S_SIM, 64), lambda i: (i, 0, 0)),
        pl.BlockSpec((1, 1, BB), lambda i: (i, 0, 0)),
        pl.BlockSpec((1, 1, BB), lambda i: (i, 0, 0)),
        full((D, D)), full((D, D)),
        full((D, D)), full((D, D)), full((D, D)), full((D, D)),
        full((D, D)), full((D, D)), full((D, D)), full((D, D)),
        full((D, D)), full((1, D)), full((D, D)), full((1, D)),
        full((4, 2, D, S_DIM)), full((4, S_DIM)), full((4, D)), full((1, 4)),
    ]
    return pl.pallas_call(
        body,
        grid=(nb,),
        in_specs=in_specs,
        out_specs=pl.BlockSpec((BB, 4), lambda i: (i, 0)),
        out_shape=jax.ShapeDtypeStruct((B, 4), jnp.float32),
    )(ue, ie, iapre, f.reshape(B, S_SIM, 64), xh.reshape(B, L_HIST, 64),
      nf.reshape(B, S_SIM, 64),
      lens_ii.reshape(nb, 1, BB), lens_uu.reshape(nb, 1, BB), *w)


def _prep_edges(ei):
    pad = EPAD - E
    src = jnp.concatenate([ei[0].astype(jnp.int32),
                           jnp.full((pad,), DUMP, jnp.int32)])
    dst = jnp.concatenate([ei[1].astype(jnp.int32),
                           jnp.full((pad,), DUMP, jnp.int32)])
    return src.reshape(EROWS, 128), dst.reshape(EROWS, 128)


def _graph_tables(table, src2d, dst2d, deg16):
    tab_pad = jnp.pad(table, ((0, NP - N), (0, 0)))
    x2 = _tc_prep(tab_pad)
    raw1 = _sc_agg(x2, src2d, dst2d)
    a1, acc1 = _tc_norm(raw1, deg16, tab_pad, final=False)
    raw2 = _sc_agg(a1, src2d, dst2d)
    return _tc_norm(raw2, deg16, acc1, final=True)


def kernel(user, item, user_edge_index, item_edge_index, item_users,
           ii_sim_users, ii_sim_lens, user_items, uu_sim_items, uu_sim_lens,
           params):
    p = params
    usrc, udst = _prep_edges(user_edge_index)
    isrc, idst = _prep_edges(item_edge_index)

    deg2 = _sc_degree(jnp.stack([udst, idst]))
    up_fin = _graph_tables(p['user_table'], usrc, udst, deg2[0])
    it_fin = _graph_tables(p['item_table'], isrc, idst, deg2[1])

    iu2d = item_users.astype(jnp.int32).reshape(-1, 128)
    ui2d = user_items.astype(jnp.int32).reshape(-1, 128)
    iiflat = ii_sim_users.astype(jnp.int32).reshape(-1)
    uuflat = uu_sim_items.astype(jnp.int32).reshape(-1)

    ue, ie, iapre, f, xh, nf = _sc_batch(
        up_fin, it_fin, user.astype(jnp.int32), item.astype(jnp.int32),
        iu2d, iiflat, ui2d, uuflat)

    w = (p['W_ii'], p['W_uu'],
         p['Wq1'], p['Wk1'], p['Wv1'], p['Wo1'],
         p['Wq2'], p['Wk2'], p['Wv2'], p['Wo2'],
         p['Wf1'], p['bf1'].reshape(1, D), p['Wf2'], p['bf2'].reshape(1, D),
         jnp.stack([p['P%d_W1' % k].reshape(2, D, S_DIM)
                    for k in (1, 2, 3, 4)]),
         jnp.stack([p['P%d_b1' % k] for k in (1, 2, 3, 4)]),
         jnp.stack([jnp.pad(p['P%d_W2' % k][:, 0], (0, D - S_DIM))
                    for k in (1, 2, 3, 4)]),
         jnp.stack([p['P%d_b2' % k] for k in (1, 2, 3, 4)]).reshape(1, 4))

    return _tc_dense(ue, ie, iapre, f, xh, nf,
                     ii_sim_lens.astype(jnp.int32),
                     uu_sim_lens.astype(jnp.int32), w)


# speedup vs baseline: 7.3898x; 1.1272x over previous
import functools

import jax
import jax.numpy as jnp
import numpy as np
from jax import lax
from jax.experimental import pallas as pl
from jax.experimental.pallas import tpu as pltpu
from jax.experimental.pallas import tpu_sc as plsc

NUM_USER = 50000
NUM_ITEM = 50000
D = 64
B = 1024
E = 800000
L_HIST = 20
S_SIM = 10
L_SIM = 20
S_DIM = 48

N = NUM_USER + 1
NP = 50176
DUMP = N
RT = NP // 16

EB = 256
ECH = EB // 128
NBLK = 196
EPAD = 16 * NBLK * EB
EROWS = EPAD // 128

DEB = 512
DECH = DEB // 128
DNBLK = 98

NEG = np.float32(-3.4e38)


def _mesh():
    return plsc.VectorSubcoreMesh(core_axis_name="c", subcore_axis_name="s")


_SC_PARAMS = pltpu.CompilerParams(use_tc_tiling_on_sc=False)


def _sc_degree(dsts2d):
    @functools.partial(
        pl.kernel, mesh=_mesh(),
        out_type=jax.ShapeDtypeStruct((2, NP, 16), jnp.float32),
        scratch_types=[
            pltpu.VMEM((DECH, 128), jnp.int32),
            pltpu.VMEM((DECH, 128), jnp.int32),
            pltpu.VMEM((DEB, 16), jnp.float32),
            pltpu.VMEM((DEB, 16), jnp.float32),
            pltpu.VMEM_SHARED((NP, 16), jnp.float32),
            pltpu.SemaphoreType.DMA,
            pltpu.SemaphoreType.DMA,
        ],
        compiler_params=_SC_PARAMS,
    )
    def k(dst_hbm, out_hbm, dbuf0, dbuf1, ones, zbuf, acc, ssem0, ssem1):
        cid = lax.axis_index("c")
        sid = lax.axis_index("s")

        def initrow(i, _):
            ones[i, :] = jnp.full((16,), 1.0, jnp.float32)
            zbuf[i, :] = jnp.full((16,), 0.0, jnp.float32)
            return 0

        lax.fori_loop(0, DEB, initrow, 0)
        r0 = sid * RT
        for t in range(6):
            pltpu.sync_copy(zbuf, acc.at[pl.ds(r0 + t * DEB, DEB)])
        pltpu.sync_copy(zbuf.at[pl.ds(0, RT - 6 * DEB)],
                        acc.at[pl.ds(r0 + 6 * DEB, RT - 6 * DEB)])
        plsc.subcore_barrier()

        def stage(g, db):
            pltpu.sync_copy(
                dst_hbm.at[cid].at[pl.ds((sid * DNBLK + g) * DECH, DECH)],
                db)

        def fire(db, sem):
            for j in range(DECH):
                pltpu.async_copy(ones.at[pl.ds(j * 128, 128)],
                                 acc.at[db.at[j]], sem, add=True)

        def drain(db, sem):
            for j in range(DECH):
                pltpu.make_async_copy(ones.at[pl.ds(j * 128, 128)],
                                      acc.at[db.at[j]], sem).wait()

        stage(0, dbuf0)
        fire(dbuf0, ssem0)

        def blk(i2, _):
            g = 2 * i2
            stage(g + 1, dbuf1)
            fire(dbuf1, ssem1)
            drain(dbuf0, ssem0)

            @pl.when(g + 2 < DNBLK)
            def _():
                stage(g + 2, dbuf0)
                fire(dbuf0, ssem0)
            drain(dbuf1, ssem1)
            return 0

        lax.fori_loop(0, DNBLK // 2, blk, 0)
        plsc.subcore_barrier()
        pltpu.sync_copy(acc.at[pl.ds(r0, RT)],
                        out_hbm.at[cid].at[pl.ds(r0, RT)])

    return k(dsts2d)


def _sc_agg(x2, src2d, dst2d):
    @functools.partial(
        pl.kernel, mesh=_mesh(),
        out_type=jax.ShapeDtypeStruct((2, NP, 32), jnp.float32),
        scratch_types=[
            pltpu.VMEM((ECH, 128), jnp.int32),
            pltpu.VMEM((ECH, 128), jnp.int32),
            pltpu.VMEM((ECH, 128), jnp.int32),
            pltpu.VMEM((ECH, 128), jnp.int32),
            pltpu.VMEM((EB, 32), jnp.float32),
            pltpu.VMEM((EB, 32), jnp.float32),
            pltpu.VMEM_SHARED((NP, 32), jnp.float32),
            pltpu.SemaphoreType.DMA,
            pltpu.SemaphoreType.DMA,
            pltpu.SemaphoreType.DMA,
            pltpu.SemaphoreType.DMA,
        ],
        compiler_params=_SC_PARAMS,
    )
    def k(x2_hbm, src_hbm, dst_hbm, out_hbm, sb0, db0, sb1, db1,
          rows0, rows1, acc, gsem0, gsem1, ssem0, ssem1):
        cid = lax.axis_index("c")
        sid = lax.axis_index("s")

        def zrow(i, _):
            rows0[i, 0:16] = jnp.full((16,), 0.0, jnp.float32)
            rows0[i, 16:32] = jnp.full((16,), 0.0, jnp.float32)
            return 0

        lax.fori_loop(0, EB, zrow, 0)
        r0 = sid * RT
        nz = RT // EB
        for t in range(nz):
            pltpu.sync_copy(rows0, acc.at[pl.ds(r0 + t * EB, EB)])
        if RT - nz * EB:
            pltpu.sync_copy(rows0.at[pl.ds(0, RT - nz * EB)],
                            acc.at[pl.ds(r0 + nz * EB, RT - nz * EB)])
        plsc.subcore_barrier()

        def stage(g, sb, db):
            base = (sid * NBLK + g) * ECH
            pltpu.sync_copy(src_hbm.at[pl.ds(base, ECH)], sb)
            pltpu.sync_copy(dst_hbm.at[pl.ds(base, ECH)], db)

        def fire_g(sb, rows, sem):
            for j in range(ECH):
                pltpu.async_copy(x2_hbm.at[cid].at[sb.at[j]],
                                 rows.at[pl.ds(j * 128, 128)], sem)

        def drain_g(sb, rows, sem):
            for j in range(ECH):
                pltpu.make_async_copy(x2_hbm.at[cid].at[sb.at[j]],
                                      rows.at[pl.ds(j * 128, 128)],
                                      sem).wait()

        def fire_s(rows, db, sem):
            for j in range(ECH):
                pltpu.async_copy(rows.at[pl.ds(j * 128, 128)],
                                 acc.at[db.at[j]], sem, add=True)

        def drain_s(rows, db, sem):
            for j in range(ECH):
                pltpu.make_async_copy(rows.at[pl.ds(j * 128, 128)],
                                      acc.at[db.at[j]], sem).wait()

        stage(0, sb0, db0)
        fire_g(sb0, rows0, gsem0)

        def blk(i2, _):
            g = 2 * i2

            @pl.when(i2 > 0)
            def _():
                drain_s(rows1, db1, ssem1)
            stage(g + 1, sb1, db1)
            fire_g(sb1, rows1, gsem1)
            drain_g(sb0, rows0, gsem0)
            fire_s(rows0, db0, ssem0)
            drain_s(rows0, db0, ssem0)

            @pl.when(g + 2 < NBLK)
            def _():
                stage(g + 2, sb0, db0)
                fire_g(sb0, rows0, gsem0)
            drain_g(sb1, rows1, gsem1)
            fire_s(rows1, db1, ssem1)
            return 0

        lax.fori_loop(0, NBLK // 2, blk, 0)
        drain_s(rows1, db1, ssem1)
        plsc.subcore_barrier()
        pltpu.sync_copy(acc.at[pl.ds(r0, RT)],
                        out_hbm.at[cid].at[pl.ds(r0, RT)])

    return k(x2, src2d, dst2d)


def _sc_batch(up, it, user, item, iu2d, iiflat, ui2d, uuflat):
    bs = B // 32

    @functools.partial(
        pl.kernel, mesh=_mesh(),
        out_type=[
            jax.ShapeDtypeStruct((B, 64), jnp.float32),
            jax.ShapeDtypeStruct((B, 64), jnp.float32),
            jax.ShapeDtypeStruct((B, 64), jnp.float32),
            jax.ShapeDtypeStruct((B * S_SIM, 64), jnp.float32),
            jax.ShapeDtypeStruct((B * L_HIST, 64), jnp.float32),
            jax.ShapeDtypeStruct((B * S_SIM, 64), jnp.float32),
        ],
        scratch_types=[
            pltpu.VMEM((bs,), jnp.int32),
            pltpu.VMEM((bs,), jnp.int32),
            pltpu.VMEM((bs * L_HIST // 128, 128), jnp.int32),
            pltpu.VMEM((bs * L_SIM * S_SIM,), jnp.int32),
            pltpu.VMEM((bs, 64), jnp.float32),
            pltpu.VMEM((bs, 64), jnp.float32),
            pltpu.VMEM((bs * L_HIST, 64), jnp.float32),
            pltpu.VMEM((L_SIM * S_SIM, 64), jnp.float32),
            pltpu.VMEM((L_SIM * S_SIM, 64), jnp.float32),
            pltpu.VMEM((bs, 64), jnp.float32),
            pltpu.VMEM((bs * S_SIM, 64), jnp.float32),
            pltpu.VMEM((bs * S_SIM, 64), jnp.float32),
            pltpu.SemaphoreType.DMA,
            pltpu.SemaphoreType.DMA,
            pltpu.SemaphoreType.DMA,
        ],
        compiler_params=_SC_PARAMS,
    )
    def k(up_hbm, it_hbm, u_hbm, i_hbm, iu_hbm, ii_hbm, ui_hbm, uu_hbm,
          ue_out, ie_out, iap_out, f_out, xh_out, nf_out,
          ubuf, ibuf, hbuf, sbuf, uerows, ierows, hrows, srows0, srows1,
          iap, fbuf, nfbuf, sem, sm0, sm1):
        cid = lax.axis_index("c")
        sid = lax.axis_index("s")
        wid = sid * 2 + cid
        gb0 = wid * bs
        hch = bs * L_HIST // 128

        pltpu.sync_copy(u_hbm.at[pl.ds(gb0, bs)], ubuf)
        pltpu.sync_copy(i_hbm.at[pl.ds(gb0, bs)], ibuf)
        pltpu.async_copy(up_hbm.at[ubuf], uerows, sem).wait()
        pltpu.async_copy(it_hbm.at[ibuf], ierows, sem).wait()

        pltpu.sync_copy(iu_hbm.at[pl.ds(wid * hch, hch)], hbuf)
        gd = [pltpu.async_copy(up_hbm.at[hbuf.at[j]],
                               hrows.at[pl.ds(j * 128, 128)], sem)
              for j in range(hch)]
        for dsc in gd:
            dsc.wait()

        def iab(b, _):
            for j in range(4):
                ue16 = uerows[b, pl.ds(j * 16, 16)]

                def lb(l, m):
                    r = hrows[b * L_HIST + l, pl.ds(j * 16, 16)]
                    return jnp.maximum(m, r * ue16)

                m = lax.fori_loop(0, L_HIST, lb,
                                  jnp.full((16,), NEG, jnp.float32))
                iap[b, pl.ds(j * 16, 16)] = m
            return 0

        lax.fori_loop(0, bs, iab, 0)

        pltpu.sync_copy(ui_hbm.at[pl.ds(wid * hch, hch)], hbuf)
        gd = [pltpu.async_copy(it_hbm.at[hbuf.at[j]],
                               hrows.at[pl.ds(j * 128, 128)], sem)
              for j in range(hch)]
        for dsc in gd:
            dsc.wait()
        pltpu.sync_copy(hrows, xh_out.at[pl.ds(gb0 * L_HIST, bs * L_HIST)])

        nsim = L_SIM * S_SIM

        def fire_sim(tab, b, rowbuf, sm):
            pltpu.async_copy(
                tab.at[sbuf.at[pl.ds(b * nsim, 128)]],
                rowbuf.at[pl.ds(0, 128)], sm)
            pltpu.async_copy(
                tab.at[sbuf.at[pl.ds(b * nsim + 128, nsim - 128)]],
                rowbuf.at[pl.ds(128, nsim - 128)], sm)

        def drain_sim(tab, b, rowbuf, sm):
            pltpu.make_async_copy(
                tab.at[sbuf.at[pl.ds(b * nsim, 128)]],
                rowbuf.at[pl.ds(0, 128)], sm).wait()
            pltpu.make_async_copy(
                tab.at[sbuf.at[pl.ds(b * nsim + 128, nsim - 128)]],
                rowbuf.at[pl.ds(128, nsim - 128)], sm).wait()

        def pool(b, rowbuf, mrows, obuf):
            def sb(s, _2):
                for j in range(4):
                    m16 = mrows[b, pl.ds(j * 16, 16)]

                    def lb(l, m):
                        r = rowbuf[s * L_SIM + l, pl.ds(j * 16, 16)]
                        return jnp.maximum(m, r * m16)

                    m = lax.fori_loop(0, L_SIM, lb,
                                      jnp.full((16,), NEG, jnp.float32))
                    obuf[b * S_SIM + s, pl.ds(j * 16, 16)] = m
                return 0

            lax.fori_loop(0, S_SIM, sb, 0)

        def sim_pass(tab, mrows, obuf):
            fire_sim(tab, 0, srows0, sm0)

            def b2loop(b2, _):
                b = 2 * b2
                fire_sim(tab, b + 1, srows1, sm1)
                drain_sim(tab, b, srows0, sm0)
                pool(b, srows0, mrows, obuf)

                @pl.when(b + 2 < bs)
                def _():
                    fire_sim(tab, b + 2, srows0, sm0)
                drain_sim(tab, b + 1, srows1, sm1)
                pool(b + 1, srows1, mrows, obuf)
                return 0

            lax.fori_loop(0, bs // 2, b2loop, 0)

        pltpu.sync_copy(ii_hbm.at[pl.ds(gb0 * nsim, bs * nsim)], sbuf)
        sim_pass(up_hbm, uerows, fbuf)
        pltpu.sync_copy(uu_hbm.at[pl.ds(gb0 * nsim, bs * nsim)], sbuf)
        sim_pass(it_hbm, ierows, nfbuf)

        pltpu.sync_copy(uerows, ue_out.at[pl.ds(gb0, bs)])
        pltpu.sync_copy(ierows, ie_out.at[pl.ds(gb0, bs)])
        pltpu.sync_copy(iap, iap_out.at[pl.ds(gb0, bs)])
        pltpu.sync_copy(fbuf, f_out.at[pl.ds(gb0 * S_SIM, bs * S_SIM)])
        pltpu.sync_copy(nfbuf, nf_out.at[pl.ds(gb0 * S_SIM, bs * S_SIM)])

    return k(up, it, user, item, iu2d, iiflat, ui2d, uuflat)


def _tc_prep(tab_pad):
    def body(x_ref, o_ref):
        x = x_ref[...]
        o_ref[0] = x[:, :32]
        o_ref[1] = x[:, 32:]

    return pl.pallas_call(
        body,
        grid=(16,),
        in_specs=[pl.BlockSpec((RT, 64), lambda i: (i, 0))],
        out_specs=pl.BlockSpec((2, RT, 32), lambda i: (0, i, 0)),
        out_shape=jax.ShapeDtypeStruct((2, NP, 32), jnp.float32),
    )(tab_pad)


def _tc_norm(raw, deg16, prev, final):
    def body(raw_ref, deg_ref, prev_ref, *out_refs):
        i = pl.program_id(0)
        raw = raw_ref[...]
        d = jnp.maximum(deg_ref[:, 0:1], 1.0)
        a0 = raw[0] / d
        a1 = raw[1] / d
        nsq = (jnp.sum(a0 * a0, axis=1, keepdims=True)
               + jnp.sum(a1 * a1, axis=1, keepdims=True))
        inv = 1.0 / jnp.maximum(jnp.sqrt(nsq), 1e-12)
        n = jnp.concatenate([a0 * inv, a1 * inv], axis=1)
        if final:
            gi = i * RT + lax.broadcasted_iota(jnp.int32, (RT, 1), 0)
            out = (prev_ref[...] + n) * jnp.float32(1.0 / 3.0)
            out_refs[0][...] = jnp.where(gi == 0, 0.0, out)
        else:
            out_refs[0][0] = a0
            out_refs[0][1] = a1
            out_refs[1][...] = prev_ref[...] + n

    if final:
        out_shape = [jax.ShapeDtypeStruct((NP, 64), jnp.float32)]
        out_specs = [pl.BlockSpec((RT, 64), lambda i: (i, 0))]
    else:
        out_shape = [jax.ShapeDtypeStruct((2, NP, 32), jnp.float32),
                     jax.ShapeDtypeStruct((NP, 64), jnp.float32)]
        out_specs = [pl.BlockSpec((2, RT, 32), lambda i: (0, i, 0)),
                     pl.BlockSpec((RT, 64), lambda i: (i, 0))]
    res = pl.pallas_call(
        body,
        grid=(16,),
        in_specs=[pl.BlockSpec((2, RT, 32), lambda i: (0, i, 0)),
                  pl.BlockSpec((RT, 16), lambda i: (i, 0)),
                  pl.BlockSpec((RT, 64), lambda i: (i, 0))],
        out_specs=out_specs,
        out_shape=out_shape,
    )(raw, deg16, prev)
    return res[0] if final else res


def _tc_dense(ue, ie, iapre, f, xh, nf, lens_ii, lens_uu, w):
    BB = 128
    SQD = np.float32(1.0 / np.sqrt(D))
    SQH = np.float32(1.0 / np.sqrt(32))

    def body(ue_ref, ie_ref, iap_ref, f_ref, xh_ref, nf_ref, li_ref, lu_ref,
             wii_ref, wuu_ref, wq1_ref, wk1_ref, wv1_ref, wo1_ref,
             wq2_ref, wk2_ref, wv2_ref, wo2_ref, wf1_ref, bf1_ref,
             wf2_ref, bf2_ref, w1s_ref, b1s_ref, w2s_ref, b2s_ref, out_ref):
        uev = ue_ref[...]
        iev = ie_ref[...]
        iap = iap_ref[...]
        fv = f_ref[...]
        nfv = nf_ref[...]
        xhv = xh_ref[...]
        li = li_ref[...].reshape(BB)
        lu = lu_ref[...].reshape(BB)

        def sim_fuse(active, fe, lens, wmat):
            act = active @ wmat
            s = jnp.sum(act[:, None, :] * fe, axis=-1) * SQD
            mask = (lax.broadcasted_iota(jnp.int32, (BB, S_SIM), 1)
                    < lens[:, None])
            s = jnp.where(mask, s, -1e9)
            a = jax.nn.softmax(s, axis=-1)
            return jnp.sum(a[:, :, None] * fe, axis=1)

        item_neigh = sim_fuse(iap, fv, li, wii_ref[...])
        ia = 0.5 * (iap + item_neigh)

        xf = xhv.reshape(BB * L_HIST, D)
        q = (xf @ wq1_ref[...]).reshape(BB, L_HIST, D)
        kk = (xf @ wk1_ref[...]).reshape(BB, L_HIST, D)
        vv = (xf @ wv1_ref[...]).reshape(BB, L_HIST, D)
        outs = []
        for h in range(2):
            qh = q[:, :, h * 32:(h + 1) * 32]
            kh = kk[:, :, h * 32:(h + 1) * 32]
            vh = vv[:, :, h * 32:(h + 1) * 32]
            sh = lax.dot_general(qh, kh, (((2,), (2,)), ((0,), (0,))),
                                 preferred_element_type=jnp.float32) * SQH
            ah = jax.nn.softmax(sh, axis=-1)
            oh = lax.dot_general(ah, vh, (((2,), (1,)), ((0,), (0,))),
                                 preferred_element_type=jnp.float32)
            outs.append(oh)
        o = jnp.concatenate(outs, axis=-1).reshape(BB * L_HIST, D)
        x_ = o @ wo1_ref[...]
        h1 = xf + x_
        x = (h1 + jnp.maximum(h1 @ wf1_ref[...] + bf1_ref[...], 0.0)
             @ wf2_ref[...] + bf2_ref[...])
        x3 = x.reshape(BB, L_HIST, D)

        q2 = iev @ wq2_ref[...]
        k2 = (x @ wk2_ref[...]).reshape(BB, L_HIST, D)
        v2 = (x @ wv2_ref[...]).reshape(BB, L_HIST, D)
        s2 = jnp.sum(q2[:, None, :] * k2, axis=-1) * SQD
        a2 = jax.nn.softmax(s2, axis=-1)
        o2 = jnp.sum(a2[:, :, None] * v2, axis=1)
        uiv = o2 @ wo2_ref[...]

        user_neigh = sim_fuse(uiv, nfv, lu, wuu_ref[...])
        ui = 0.5 * (uiv + user_neigh)

        lefts = (uev, ui, uev, ui)
        rights = (iev, iev, ia, ia)
        cols = []
        for kq in range(4):
            hh = (lefts[kq] @ w1s_ref[kq, 0] + rights[kq] @ w1s_ref[kq, 1]
                  + b1s_ref[kq][None, :])
            hh = jnp.maximum(hh, 0.0)
            sc = jnp.sum(hh * w2s_ref[kq, :S_DIM][None, :], axis=1,
                         keepdims=True)
            cols.append(sc)
        out_ref[...] = jnp.concatenate(cols, axis=1) + b2s_ref[...]

    nb = B // BB
    full = lambda shape: pl.BlockSpec(shape, lambda i: tuple(0 for _ in shape))
    in_specs = [
        pl.BlockSpec((BB, 64), lambda i: (i, 0)),
        pl.BlockSpec((BB, 64), lambda i: (i, 0)),
        pl.BlockSpec((BB, 64), lambda i: (i, 0)),
        pl.BlockSpec((BB, S_SIM, 64), lambda i: (i, 0, 0)),
        pl.BlockSpec((BB, L_HIST, 64), lambda i: (i, 0, 0)),
        pl.BlockSpec((BB, S_SIM, 64), lambda i: (i, 0, 0)),
        pl.BlockSpec((1, 1, BB), lambda i: (i, 0, 0)),
        pl.BlockSpec((1, 1, BB), lambda i: (i, 0, 0)),
        full((D, D)), full((D, D)),
        full((D, D)), full((D, D)), full((D, D)), full((D, D)),
        full((D, D)), full((D, D)), full((D, D)), full((D, D)),
        full((D, D)), full((1, D)), full((D, D)), full((1, D)),
        full((4, 2, D, S_DIM)), full((4, S_DIM)), full((4, D)), full((1, 4)),
    ]
    return pl.pallas_call(
        body,
        grid=(nb,),
        in_specs=in_specs,
        out_specs=pl.BlockSpec((BB, 4), lambda i: (i, 0)),
        out_shape=jax.ShapeDtypeStruct((B, 4), jnp.float32),
    )(ue, ie, iapre, f.reshape(B, S_SIM, 64), xh.reshape(B, L_HIST, 64),
      nf.reshape(B, S_SIM, 64),
      lens_ii.reshape(nb, 1, BB), lens_uu.reshape(nb, 1, BB), *w)


def _prep_edges(ei):
    pad = EPAD - E
    src = jnp.concatenate([ei[0].astype(jnp.int32),
                           jnp.full((pad,), DUMP, jnp.int32)])
    dst = jnp.concatenate([ei[1].astype(jnp.int32),
                           jnp.full((pad,), DUMP, jnp.int32)])
    return src.reshape(EROWS, 128), dst.reshape(EROWS, 128)


def _graph_tables(table, src2d, dst2d, deg16):
    tab_pad = jnp.pad(table, ((0, NP - N), (0, 0)))
    x2 = _tc_prep(tab_pad)
    raw1 = _sc_agg(x2, src2d, dst2d)
    a1, acc1 = _tc_norm(raw1, deg16, tab_pad, final=False)
    raw2 = _sc_agg(a1, src2d, dst2d)
    return _tc_norm(raw2, deg16, acc1, final=True)


def kernel(user, item, user_edge_index, item_edge_index, item_users,
           ii_sim_users, ii_sim_lens, user_items, uu_sim_items, uu_sim_lens,
           params):
    p = params
    usrc, udst = _prep_edges(user_edge_index)
    isrc, idst = _prep_edges(item_edge_index)

    deg2 = _sc_degree(jnp.stack([udst, idst]))
    up_fin = _graph_tables(p['user_table'], usrc, udst, deg2[0])
    it_fin = _graph_tables(p['item_table'], isrc, idst, deg2[1])

    iu2d = item_users.astype(jnp.int32).reshape(-1, 128)
    ui2d = user_items.astype(jnp.int32).reshape(-1, 128)
    iiflat = ii_sim_users.astype(jnp.int32).reshape(-1)
    uuflat = uu_sim_items.astype(jnp.int32).reshape(-1)

    ue, ie, iapre, f, xh, nf = _sc_batch(
        up_fin, it_fin, user.astype(jnp.int32), item.astype(jnp.int32),
        iu2d, iiflat, ui2d, uuflat)

    w = (p['W_ii'], p['W_uu'],
         p['Wq1'], p['Wk1'], p['Wv1'], p['Wo1'],
         p['Wq2'], p['Wk2'], p['Wv2'], p['Wo2'],
         p['Wf1'], p['bf1'].reshape(1, D), p['Wf2'], p['bf2'].reshape(1, D),
         jnp.stack([p['P%d_W1' % k].reshape(2, D, S_DIM)
                    for k in (1, 2, 3, 4)]),
         jnp.stack([p['P%d_b1' % k] for k in (1, 2, 3, 4)]),
         jnp.stack([jnp.pad(p['P%d_W2' % k][:, 0], (0, D - S_DIM))
                    for k in (1, 2, 3, 4)]),
         jnp.stack([p['P%d_b2' % k] for k in (1, 2, 3, 4)]).reshape(1, 4))

    return _tc_dense(ue, ie, iapre, f, xh, nf,
                     ii_sim_lens.astype(jnp.int32),
                     uu_sim_lens.astype(jnp.int32), w)
